# baseline (speedup 1.0000x reference)
import functools

import jax
import jax.numpy as jnp
import numpy as np
from jax import lax
from jax.experimental import pallas as pl
from jax.experimental.pallas import tpu as pltpu
from jax.experimental.pallas import tpu_sc as plsc

N_USERS = 1000000
N_ITEMS = 1000000
DIM = 16
B = 16384
L = 50

NC = 2
NS = 16
NW = NC * NS
UPW = B // NW
CU = 64
NCHUNK = UPW // CU
CI = CU * L
NG = CI // DIM


def _gmf_body(uidx_hbm, iidx_hbm, utab_hbm, itab_hbm, w_hbm, bias_hbm, uidl_hbm,
              out_hbm,
              uidx_v, iidx_v, urows_v, irows_v, uidl_v, out_v, wv, biasv,
              sem_u, sem_i):
    wid = lax.axis_index("s") * NC + lax.axis_index("c")

    pltpu.sync_copy(w_hbm, wv)
    pltpu.sync_copy(bias_hbm, biasv)
    pltpu.sync_copy(uidl_hbm, uidl_v)

    iota = lax.iota(jnp.int32, DIM)
    fds = [jnp.full((DIM,), d, dtype=jnp.int32) for d in range(DIM)]
    wsplat = [wv[d, :] for d in range(DIM)]
    bias_vec = biasv[...]

    def chunk_body(c, _):
        ubase = pl.multiple_of(wid * UPW + c * CU, CU)
        ibase = pl.multiple_of(ubase * L, CI)
        pltpu.sync_copy(uidx_hbm.at[pl.ds(ubase, CU)], uidx_v)
        pltpu.sync_copy(iidx_hbm.at[pl.ds(ibase, CI)], iidx_v)
        cp_u = pltpu.async_copy(utab_hbm.at[uidx_v], urows_v, sem_u)
        cp_i = pltpu.async_copy(itab_hbm.at[iidx_v], irows_v, sem_i)
        cp_u.wait()
        cp_i.wait()

        def group_body(g, _):
            gb = pl.multiple_of(g * DIM, DIM)
            pos = gb + iota
            uid = uidl_v[pl.ds(gb, DIM)]
            acc = bias_vec
            for d in range(DIM):
                col = plsc.load_gather(irows_v, [pos, fds[d]])
                pd = plsc.load_gather(urows_v, [uid, fds[d]])
                acc = acc + col * pd * wsplat[d]
            out_v[pl.ds(gb, DIM)] = acc
            return 0

        lax.fori_loop(0, NG, group_body, 0)
        pltpu.sync_copy(out_v, out_hbm.at[pl.ds(ibase, CI)])
        return 0

    lax.fori_loop(0, NCHUNK, chunk_body, 0)


@jax.jit
def _gmf(user_indices, item_idx_flat, user_table, item_table, w16, bias16, uidl):
    mesh = plsc.VectorSubcoreMesh(core_axis_name="c", subcore_axis_name="s")
    kfn = pl.kernel(
        _gmf_body,
        out_type=jax.ShapeDtypeStruct((B * L,), jnp.float32),
        mesh=mesh,
        compiler_params=pltpu.CompilerParams(
            needs_layout_passes=False, use_tc_tiling_on_sc=False),
        scratch_types=[
            pltpu.VMEM((CU,), jnp.int32),
            pltpu.VMEM((CI,), jnp.int32),
            pltpu.VMEM((CU, DIM), jnp.float32),
            pltpu.VMEM((CI, DIM), jnp.float32),
            pltpu.VMEM((CI,), jnp.int32),
            pltpu.VMEM((CI,), jnp.float32),
            pltpu.VMEM((DIM, DIM), jnp.float32),
            pltpu.VMEM((DIM,), jnp.float32),
            pltpu.SemaphoreType.DMA,
            pltpu.SemaphoreType.DMA,
        ],
    )
    return kfn(user_indices, item_idx_flat, user_table, item_table, w16, bias16, uidl)


def kernel(user_indices, item_indices, user_table, item_table, W, b):
    item_idx_flat = item_indices.reshape(B * L)
    w16 = jnp.broadcast_to(W.reshape(DIM, 1), (DIM, DIM))
    bias16 = jnp.broadcast_to(b, (DIM,))
    uidl = jnp.asarray(np.arange(CI, dtype=np.int32) // L)
    out = _gmf(user_indices, item_idx_flat, user_table, item_table, w16, bias16, uidl)
    return out.reshape(B, L, 1)
